# Initial kernel scaffold; baseline (speedup 1.0000x reference)
import functools

import jax
import jax.numpy as jnp
from jax import lax
from jax.experimental import pallas as pl
from jax.experimental.pallas import tpu as pltpu
from jax.experimental.pallas import tpu_sc as plsc

N_NODES = 100000
N_EDGES = 1600000

NC = 2
NS = 16
NW = NC * NS
LANE = 128

NP = 100096
ROWS_PER_TILE_OUT = NP // NS

EROWS = -(-N_EDGES // (NW * LANE)) * NW
EP = EROWS * LANE
ROWS_PER_W = EROWS // NW
T_UNROLL = 23
N_OUTER = ROWS_PER_W // T_UNROLL

TC_R = 5888
TC_GRID = NP // TC_R


def _worker_id():
    return lax.axis_index("c") * NS + lax.axis_index("s")


def _deg_body(dstp, z1, out, didx, ones, acc, gsem):
    c = lax.axis_index("c")
    s = lax.axis_index("s")
    for j in range(LANE // 16):
        ones[pl.ds(j * 16, 16)] = jnp.ones((16,), jnp.float32)
    pltpu.sync_copy(z1, acc.at[pl.ds(s * ROWS_PER_TILE_OUT, ROWS_PER_TILE_OUT)])
    plsc.subcore_barrier()

    base_row = _worker_id() * ROWS_PER_W

    def step(i, _):
        row0 = base_row + i * T_UNROLL
        pltpu.sync_copy(dstp.at[pl.ds(row0, T_UNROLL)], didx)
        descs = [
            pltpu.async_copy(ones, acc.at[didx.at[j]], gsem, add=True)
            for j in range(T_UNROLL)
        ]
        for d in descs:
            d.wait()
        return ()

    lax.fori_loop(0, N_OUTER, step, ())
    plsc.subcore_barrier()
    sl = pl.ds(s * ROWS_PER_TILE_OUT, ROWS_PER_TILE_OUT)
    pltpu.sync_copy(acc.at[sl], out.at[c, sl])


_deg_kernel = functools.partial(
    pl.kernel,
    out_type=jax.ShapeDtypeStruct((NC, NP), jnp.float32),
    mesh=plsc.VectorSubcoreMesh(core_axis_name="c", subcore_axis_name="s"),
    scratch_types=[
        pltpu.VMEM((T_UNROLL, LANE), jnp.int32),
        pltpu.VMEM((LANE,), jnp.float32),
        pltpu.VMEM_SHARED((NP,), jnp.float32),
        pltpu.SemaphoreType.DMA,
    ],
)(_deg_body)


def _agg_body(g, srcp, dstp, zf, out, sidx, didx, rows, acc, gsem, ssem):
    c = lax.axis_index("c")
    s = lax.axis_index("s")
    pltpu.sync_copy(zf, acc.at[pl.ds(s * ROWS_PER_TILE_OUT, ROWS_PER_TILE_OUT)])
    plsc.subcore_barrier()

    base_row = _worker_id() * ROWS_PER_W

    def step(i, _):
        row0 = base_row + i * T_UNROLL
        pltpu.sync_copy(srcp.at[pl.ds(row0, T_UNROLL)], sidx)
        pltpu.sync_copy(dstp.at[pl.ds(row0, T_UNROLL)], didx)
        gds = [
            pltpu.async_copy(g.at[sidx.at[j]], rows.at[j], gsem)
            for j in range(T_UNROLL)
        ]
        for d in gds:
            d.wait()
        sds = [
            pltpu.async_copy(rows.at[j], acc.at[didx.at[j]], ssem, add=True)
            for j in range(T_UNROLL)
        ]
        for d in sds:
            d.wait()
        return ()

    lax.fori_loop(0, N_OUTER, step, ())
    plsc.subcore_barrier()
    sl = pl.ds(s * ROWS_PER_TILE_OUT, ROWS_PER_TILE_OUT)
    pltpu.sync_copy(acc.at[sl], out.at[c, sl])


def _make_agg(F):
    return functools.partial(
        pl.kernel,
        out_type=jax.ShapeDtypeStruct((NC, NP, F), jnp.float32),
        mesh=plsc.VectorSubcoreMesh(core_axis_name="c", subcore_axis_name="s"),
        scratch_types=[
            pltpu.VMEM((T_UNROLL, LANE), jnp.int32),
            pltpu.VMEM((T_UNROLL, LANE), jnp.int32),
            pltpu.VMEM((T_UNROLL, LANE, F), jnp.float32),
            pltpu.VMEM_SHARED((NP, F), jnp.float32),
            pltpu.SemaphoreType.DMA,
            pltpu.SemaphoreType.DMA,
        ],
    )(_agg_body)


_agg4 = _make_agg(4)
_agg2 = _make_agg(2)


def _tc0_body(degp, x, w1, dinv_o, g1_o):
    deg = degp[0] + degp[1] + 1.0
    dinv = lax.rsqrt(deg)
    dinv_o[...] = dinv
    h = jnp.dot(x[...], w1[...], preferred_element_type=jnp.float32)
    g1_o[...] = dinv[:, None] * h


def _tc0(degp, x, w1):
    return pl.pallas_call(
        _tc0_body,
        grid=(TC_GRID,),
        in_specs=[
            pl.BlockSpec((NC, TC_R), lambda i: (0, i)),
            pl.BlockSpec((TC_R, 34), lambda i: (i, 0)),
            pl.BlockSpec((34, 4), lambda i: (0, 0)),
        ],
        out_specs=[
            pl.BlockSpec((TC_R,), lambda i: (i,)),
            pl.BlockSpec((TC_R, 4), lambda i: (i, 0)),
        ],
        out_shape=[
            jax.ShapeDtypeStruct((NP,), jnp.float32),
            jax.ShapeDtypeStruct((NP, 4), jnp.float32),
        ],
    )(degp, x, w1)


def _tcmid_body(accp, gk, dinv, b, w, gn_o):
    dv = dinv[...]
    hk = jax.nn.relu(dv[:, None] * (accp[0] + accp[1] + gk[...]) + b[...])
    gn_o[...] = dv[:, None] * jnp.dot(hk, w[...], preferred_element_type=jnp.float32)


def _tcmid(accp, gk, dinv, b, w):
    F = gk.shape[1]
    Fn = w.shape[1]
    return pl.pallas_call(
        _tcmid_body,
        grid=(TC_GRID,),
        in_specs=[
            pl.BlockSpec((NC, TC_R, F), lambda i: (0, i, 0)),
            pl.BlockSpec((TC_R, F), lambda i: (i, 0)),
            pl.BlockSpec((TC_R,), lambda i: (i,)),
            pl.BlockSpec((1, F), lambda i: (0, 0)),
            pl.BlockSpec((F, Fn), lambda i: (0, 0)),
        ],
        out_specs=pl.BlockSpec((TC_R, Fn), lambda i: (i, 0)),
        out_shape=jax.ShapeDtypeStruct((NP, Fn), jnp.float32),
    )(accp, gk, dinv, b, w)


def _tcfin_body(accp, g3, dinv, b3, wc, bc, out_o):
    dv = dinv[...]
    h3 = jax.nn.relu(dv[:, None] * (accp[0] + accp[1] + g3[...]) + b3[...])
    logits = jnp.dot(h3, wc[...], preferred_element_type=jnp.float32) + bc[...]
    m = jnp.max(logits, axis=1, keepdims=True)
    z = logits - m
    out_o[...] = z - jnp.log(jnp.sum(jnp.exp(z), axis=1, keepdims=True))


def _tcfin(accp, g3, dinv, b3, wc, bc):
    return pl.pallas_call(
        _tcfin_body,
        grid=(TC_GRID,),
        in_specs=[
            pl.BlockSpec((NC, TC_R, 2), lambda i: (0, i, 0)),
            pl.BlockSpec((TC_R, 2), lambda i: (i, 0)),
            pl.BlockSpec((TC_R,), lambda i: (i,)),
            pl.BlockSpec((1, 2), lambda i: (0, 0)),
            pl.BlockSpec((2, 4), lambda i: (0, 0)),
            pl.BlockSpec((1, 4), lambda i: (0, 0)),
        ],
        out_specs=pl.BlockSpec((TC_R, 4), lambda i: (i, 0)),
        out_shape=jax.ShapeDtypeStruct((NP, 4), jnp.float32),
    )(accp, g3, dinv, b3, wc, bc)


def kernel(x, edge_index, W1, b1, W2, b2, W3, b3, Wc, bc):
    f32 = jnp.float32
    src = edge_index[0]
    dst = edge_index[1]
    srcp = jnp.full((EP,), N_NODES, jnp.int32).at[:N_EDGES].set(src)
    dstp = jnp.full((EP,), N_NODES, jnp.int32).at[:N_EDGES].set(dst)
    srcp = srcp.reshape(EROWS, LANE)
    dstp = dstp.reshape(EROWS, LANE)
    x_pad = jnp.zeros((NP, 34), f32).at[:N_NODES].set(x)

    z1 = jnp.zeros((ROWS_PER_TILE_OUT,), f32)
    z4 = jnp.zeros((ROWS_PER_TILE_OUT, 4), f32)
    z2 = jnp.zeros((ROWS_PER_TILE_OUT, 2), f32)

    degp = _deg_kernel(dstp, z1)
    dinv, g1 = _tc0(degp, x_pad, W1)
    acc1 = _agg4(g1, srcp, dstp, z4)
    g2 = _tcmid(acc1, g1, dinv, b1.reshape(1, 4), W2)
    acc2 = _agg4(g2, srcp, dstp, z4)
    g3 = _tcmid(acc2, g2, dinv, b2.reshape(1, 4), W3)
    acc3 = _agg2(g3, srcp, dstp, z2)
    out = _tcfin(acc3, g3, dinv, b3.reshape(1, 2), Wc, bc.reshape(1, 4))
    return out[:N_NODES]


# speedup vs baseline: 35.9064x; 35.9064x over previous
import functools

import jax
import jax.numpy as jnp
from jax import lax
from jax.experimental import pallas as pl
from jax.experimental.pallas import tpu as pltpu
from jax.experimental.pallas import tpu_sc as plsc

N_NODES = 100000
N_EDGES = 1600000

NC = 2
NS = 16
NW = NC * NS
LANE = 128
F = 8

NP = 100096
RPT = NP // NS

ROWS_PER_W = 392
EROWS = NW * ROWS_PER_W
EP = EROWS * LANE
T_UNROLL = 8
N_OUTER = ROWS_PER_W // T_UNROLL

TC_R = 5888
TC_GRID = NP // TC_R

_SC_PARAMS = pltpu.CompilerParams(use_tc_tiling_on_sc=False)


def _worker_id():
    return lax.axis_index("c") * NS + lax.axis_index("s")


def _deg_body(dstp, z1, out, didx, ones, stg, acc, gsem):
    c = lax.axis_index("c")
    s = lax.axis_index("s")
    for j in range(LANE // 16):
        ones[pl.ds(j * 16, 16)] = jnp.ones((16,), jnp.float32)
    pltpu.sync_copy(z1, stg)
    pltpu.sync_copy(stg, acc.at[pl.ds(s * RPT, RPT)])
    plsc.subcore_barrier()

    base_row = _worker_id() * ROWS_PER_W

    def step(i, _):
        row0 = base_row + i * T_UNROLL
        pltpu.sync_copy(dstp.at[pl.ds(row0, T_UNROLL)], didx)
        descs = [
            pltpu.async_copy(ones, acc.at[didx.at[j]], gsem, add=True)
            for j in range(T_UNROLL)
        ]
        for d in descs:
            d.wait()
        return ()

    lax.fori_loop(0, N_OUTER, step, ())
    plsc.subcore_barrier()
    sl = pl.ds(s * RPT, RPT)
    pltpu.sync_copy(acc.at[sl], stg)
    pltpu.sync_copy(stg, out.at[pl.ds(c * NP + s * RPT, RPT)])


_deg_kernel = functools.partial(
    pl.kernel,
    out_type=jax.ShapeDtypeStruct((NC * NP,), jnp.float32),
    mesh=plsc.VectorSubcoreMesh(core_axis_name="c", subcore_axis_name="s"),
    compiler_params=_SC_PARAMS,
    scratch_types=[
        pltpu.VMEM((T_UNROLL, LANE), jnp.int32),
        pltpu.VMEM((LANE,), jnp.float32),
        pltpu.VMEM((RPT,), jnp.float32),
        pltpu.VMEM_SHARED((NP,), jnp.float32),
        pltpu.SemaphoreType.DMA,
    ],
)(_deg_body)


def _agg_body(g, srcp, dstp, zf, out, sidx, didx, rows, stg, acc, gsem, ssem):
    c = lax.axis_index("c")
    s = lax.axis_index("s")
    pltpu.sync_copy(zf, stg)
    pltpu.sync_copy(stg, acc.at[pl.ds(s * RPT, RPT)])
    plsc.subcore_barrier()

    base_row = _worker_id() * ROWS_PER_W

    def step(i, _):
        row0 = base_row + i * T_UNROLL
        pltpu.sync_copy(srcp.at[pl.ds(row0, T_UNROLL)], sidx)
        pltpu.sync_copy(dstp.at[pl.ds(row0, T_UNROLL)], didx)
        gds = [
            pltpu.async_copy(g.at[sidx.at[j]], rows.at[j], gsem)
            for j in range(T_UNROLL)
        ]
        for d in gds:
            d.wait()
        sds = [
            pltpu.async_copy(rows.at[j], acc.at[didx.at[j]], ssem, add=True)
            for j in range(T_UNROLL)
        ]
        for d in sds:
            d.wait()
        return ()

    lax.fori_loop(0, N_OUTER, step, ())
    plsc.subcore_barrier()
    sl = pl.ds(s * RPT, RPT)
    pltpu.sync_copy(acc.at[sl], stg)
    pltpu.sync_copy(stg, out.at[c, sl])


_agg_kernel = functools.partial(
    pl.kernel,
    out_type=jax.ShapeDtypeStruct((NC, NP, F), jnp.float32),
    mesh=plsc.VectorSubcoreMesh(core_axis_name="c", subcore_axis_name="s"),
    compiler_params=_SC_PARAMS,
    scratch_types=[
        pltpu.VMEM((T_UNROLL, LANE), jnp.int32),
        pltpu.VMEM((T_UNROLL, LANE), jnp.int32),
        pltpu.VMEM((T_UNROLL, LANE, F), jnp.float32),
        pltpu.VMEM((RPT, F), jnp.float32),
        pltpu.VMEM_SHARED((NP, F), jnp.float32),
        pltpu.SemaphoreType.DMA,
        pltpu.SemaphoreType.DMA,
    ],
)(_agg_body)


def _tc0_body(degp, x, w1, dinv_o, g1_o):
    deg = degp[0] + degp[1] + 1.0
    dinv = lax.rsqrt(deg)
    dinv_o[...] = dinv[:, None]
    h = jnp.dot(x[...], w1[...], preferred_element_type=jnp.float32)
    g1_o[...] = dinv[:, None] * h


def _tc0(degp, x, w1):
    return pl.pallas_call(
        _tc0_body,
        grid=(TC_GRID,),
        in_specs=[
            pl.BlockSpec((NC, TC_R), lambda i: (0, i)),
            pl.BlockSpec((TC_R, 34), lambda i: (i, 0)),
            pl.BlockSpec((34, F), lambda i: (0, 0)),
        ],
        out_specs=[
            pl.BlockSpec((TC_R, 1), lambda i: (i, 0)),
            pl.BlockSpec((TC_R, F), lambda i: (i, 0)),
        ],
        out_shape=[
            jax.ShapeDtypeStruct((NP, 1), jnp.float32),
            jax.ShapeDtypeStruct((NP, F), jnp.float32),
        ],
    )(degp, x, w1)


def _tcmid_body(accp, gk, dinv, b, w, gn_o):
    dv = dinv[...]
    hk = jax.nn.relu(dv * (accp[0] + accp[1] + gk[...]) + b[...])
    gn_o[...] = dv * jnp.dot(hk, w[...], preferred_element_type=jnp.float32)


def _tcmid(accp, gk, dinv, b, w):
    return pl.pallas_call(
        _tcmid_body,
        grid=(TC_GRID,),
        in_specs=[
            pl.BlockSpec((NC, TC_R, F), lambda i: (0, i, 0)),
            pl.BlockSpec((TC_R, F), lambda i: (i, 0)),
            pl.BlockSpec((TC_R, 1), lambda i: (i, 0)),
            pl.BlockSpec((1, F), lambda i: (0, 0)),
            pl.BlockSpec((F, F), lambda i: (0, 0)),
        ],
        out_specs=pl.BlockSpec((TC_R, F), lambda i: (i, 0)),
        out_shape=jax.ShapeDtypeStruct((NP, F), jnp.float32),
    )(accp, gk, dinv, b, w)


def _tcfin_body(accp, g3, dinv, b3, wc, bc, out_o):
    dv = dinv[...]
    h3 = jax.nn.relu(dv * (accp[0] + accp[1] + g3[...]) + b3[...])
    logits = jnp.dot(h3, wc[...], preferred_element_type=jnp.float32) + bc[...]
    m = jnp.max(logits, axis=1, keepdims=True)
    z = logits - m
    out_o[...] = z - jnp.log(jnp.sum(jnp.exp(z), axis=1, keepdims=True))


def _tcfin(accp, g3, dinv, b3, wc, bc):
    return pl.pallas_call(
        _tcfin_body,
        grid=(TC_GRID,),
        in_specs=[
            pl.BlockSpec((NC, TC_R, F), lambda i: (0, i, 0)),
            pl.BlockSpec((TC_R, F), lambda i: (i, 0)),
            pl.BlockSpec((TC_R, 1), lambda i: (i, 0)),
            pl.BlockSpec((1, F), lambda i: (0, 0)),
            pl.BlockSpec((F, 4), lambda i: (0, 0)),
            pl.BlockSpec((1, 4), lambda i: (0, 0)),
        ],
        out_specs=pl.BlockSpec((TC_R, 4), lambda i: (i, 0)),
        out_shape=jax.ShapeDtypeStruct((NP, 4), jnp.float32),
    )(accp, g3, dinv, b3, wc, bc)


def _padw(w, r, c):
    return jnp.zeros((r, c), jnp.float32).at[: w.shape[0], : w.shape[1]].set(w)


def kernel(x, edge_index, W1, b1, W2, b2, W3, b3, Wc, bc):
    f32 = jnp.float32
    src = edge_index[0]
    dst = edge_index[1]
    srcp = jnp.full((EP,), N_NODES, jnp.int32).at[:N_EDGES].set(src)
    dstp = jnp.full((EP,), N_NODES, jnp.int32).at[:N_EDGES].set(dst)
    srcp = srcp.reshape(EROWS, LANE)
    dstp = dstp.reshape(EROWS, LANE)
    x_pad = jnp.zeros((NP, 34), f32).at[:N_NODES].set(x)

    z1 = jnp.zeros((RPT,), f32)
    zf = jnp.zeros((RPT, F), f32)

    w1p = _padw(W1, 34, F)
    w2p = _padw(W2, F, F)
    w3p = _padw(W3, F, F)
    wcp = _padw(Wc, F, 4)
    b1p = _padw(b1.reshape(1, -1), 1, F)
    b2p = _padw(b2.reshape(1, -1), 1, F)
    b3p = _padw(b3.reshape(1, -1), 1, F)

    degp = _deg_kernel(dstp, z1).reshape(NC, NP)
    dinv, g1 = _tc0(degp, x_pad, w1p)
    acc1 = _agg_kernel(g1, srcp, dstp, zf)
    g2 = _tcmid(acc1, g1, dinv, b1p, w2p)
    acc2 = _agg_kernel(g2, srcp, dstp, zf)
    g3 = _tcmid(acc2, g2, dinv, b2p, w3p)
    acc3 = _agg_kernel(g3, srcp, dstp, zf)
    out = _tcfin(acc3, g3, dinv, b3p, wcp, bc.reshape(1, 4))
    return out[:N_NODES]
